# baseline (speedup 1.0000x reference)
import functools

import jax
import jax.numpy as jnp
from jax import lax
from jax.experimental import pallas as pl
from jax.experimental.pallas import tpu as pltpu
from jax.experimental.pallas import tpu_sc as plsc

N = 10000
E = 320000
D = 128
C = 40
DP = 48
NC = 2
NS = 16
NW = NC * NS
EPW = E // NW
CH = 128
NCH = 80
EPWP = CH * NCH
NPAD = N + 8
RING = 8
GDEPTH = 6
NPS = N // NS
BR = 1000
NB = N // BR

_MESH = plsc.VectorSubcoreMesh(core_axis_name="c", subcore_axis_name="s")
_SC_PARAMS = pltpu.CompilerParams(use_tc_tiling_on_sc=False)


def _deg_body(dst_hbm, ones_hbm, zeros_hbm, out_hbm, idx_d, ones_v, acc,
              sem, dsem):
    c = lax.axis_index("c")
    s = lax.axis_index("s")

    pltpu.sync_copy(zeros_hbm.at[pl.ds(s * NPS, NPS)],
                    acc.at[pl.ds(s * NPS, NPS)])
    pltpu.sync_copy(ones_hbm, ones_v)
    wid = s * NC + c
    pltpu.sync_copy(dst_hbm.at[wid], idx_d)
    plsc.subcore_barrier()

    @pl.loop(0, NCH, step=8)
    def _(j):
        for b in range(8):
            pltpu.async_copy(ones_v, acc.at[idx_d.at[j + b]], sem, add=True)
        for b in range(8):
            pltpu.make_async_copy(ones_v, acc.at[idx_d.at[j + b]], sem).wait()

    plsc.subcore_barrier()
    pltpu.sync_copy(acc.at[pl.ds(s * NPS, NPS)],
                    out_hbm.at[c, pl.ds(s * NPS, NPS)])


@functools.partial(
    pl.kernel,
    out_type=jax.ShapeDtypeStruct((NC, N, 16), jnp.float32),
    mesh=_MESH,
    scratch_types=[
        pltpu.VMEM((NCH, CH), jnp.int32),
        pltpu.VMEM((CH, 16), jnp.float32),
        pltpu.VMEM_SHARED((NPAD, 16), jnp.float32),
        pltpu.SemaphoreType.DMA,
        pltpu.SemaphoreType.DMA,
    ],
    compiler_params=_SC_PARAMS,
)
def _deg_kernel(dst_hbm, ones_hbm, zeros_hbm, out_hbm, idx_d, ones_v, acc,
                sem, dsem):
    _deg_body(dst_hbm, ones_hbm, zeros_hbm, out_hbm, idx_d, ones_v, acc,
              sem, dsem)


def _hop_body(g_hbm, src_hbm, dst_hbm, zeros_hbm, out_hbm,
              idx_s, idx_d, rows, acc, gsems, ssems, dsem):
    c = lax.axis_index("c")
    s = lax.axis_index("s")

    @pl.when(c == 0)
    def _():
        pltpu.sync_copy(g_hbm.at[pl.ds(s * NPS, NPS)],
                        acc.at[pl.ds(s * NPS, NPS)])

    @pl.when(c != 0)
    def _():
        pltpu.sync_copy(zeros_hbm.at[pl.ds(s * NPS, NPS)],
                        acc.at[pl.ds(s * NPS, NPS)])

    wid = s * NC + c
    pltpu.sync_copy(src_hbm.at[wid], idx_s)
    pltpu.sync_copy(dst_hbm.at[wid], idx_d)
    plsc.subcore_barrier()

    for m in range(GDEPTH):
        pltpu.async_copy(g_hbm.at[idx_s.at[m]], rows[m], gsems[m])

    @pl.loop(0, NCH, step=RING)
    def _(j):
        for r in range(RING):
            k = j + r
            b2 = (r + GDEPTH) % RING

            @pl.when((k >= RING - GDEPTH) & (k + GDEPTH < NCH))
            def _():
                pltpu.make_async_copy(rows[b2], acc.at[idx_d.at[k]],
                                      ssems[b2]).wait()

            @pl.when(k + GDEPTH < NCH)
            def _():
                pltpu.async_copy(g_hbm.at[idx_s.at[k + GDEPTH]], rows[b2],
                                 gsems[b2])

            pltpu.make_async_copy(g_hbm.at[idx_s.at[k]], rows[r],
                                  gsems[r]).wait()
            pltpu.async_copy(rows[r], acc.at[idx_d.at[k]], ssems[r],
                             add=True)

    for r in range(RING):
        pltpu.make_async_copy(rows[r], acc.at[idx_d.at[r]], ssems[r]).wait()

    plsc.subcore_barrier()
    pltpu.sync_copy(acc.at[pl.ds(s * NPS, NPS)],
                    out_hbm.at[c, pl.ds(s * NPS, NPS)])


@functools.partial(
    pl.kernel,
    out_type=jax.ShapeDtypeStruct((NC, N, DP), jnp.float32),
    mesh=_MESH,
    scratch_types=[
        pltpu.VMEM((NCH, CH), jnp.int32),
        pltpu.VMEM((NCH, CH), jnp.int32),
        [pltpu.VMEM((CH, DP), jnp.float32)] * RING,
        pltpu.VMEM_SHARED((NPAD, DP), jnp.float32),
        [pltpu.SemaphoreType.DMA] * RING,
        [pltpu.SemaphoreType.DMA] * RING,
        pltpu.SemaphoreType.DMA,
    ],
    compiler_params=_SC_PARAMS,
)
def _hop_kernel(g_hbm, src_hbm, dst_hbm, zeros_hbm, out_hbm,
                idx_s, idx_d, rows, acc, gsems, ssems, dsem):
    _hop_body(g_hbm, src_hbm, dst_hbm, zeros_hbm, out_hbm,
              idx_s, idx_d, rows, acc, gsems, ssems, dsem)


def _mm_body(x_ref, w_ref, o_ref):
    o_ref[...] = lax.dot_general(
        x_ref[...], w_ref[...], (((1,), (1,)), ((), ())),
        preferred_element_type=jnp.float32)


def _matmul(x, wp):
    return pl.pallas_call(
        _mm_body,
        grid=(NB,),
        in_specs=[
            pl.BlockSpec((BR, D), lambda i: (i, 0)),
            pl.BlockSpec((DP, D), lambda i: (0, 0)),
        ],
        out_specs=pl.BlockSpec((BR, DP), lambda i: (i, 0)),
        out_shape=jax.ShapeDtypeStruct((N, DP), jnp.float32),
    )(x, wp)


def _scale1_body(z_ref, d0_ref, d1_ref, o_ref):
    deg = 1.0 + d0_ref[0, :, 0:1] + d1_ref[0, :, 0:1]
    o_ref[...] = z_ref[...] * lax.rsqrt(deg)


def _scale1(z, degp):
    return pl.pallas_call(
        _scale1_body,
        grid=(NB,),
        in_specs=[
            pl.BlockSpec((BR, DP), lambda i: (i, 0)),
            pl.BlockSpec((1, BR, 16), lambda i: (0, i, 0)),
            pl.BlockSpec((1, BR, 16), lambda i: (1, i, 0)),
        ],
        out_specs=pl.BlockSpec((BR, DP), lambda i: (i, 0)),
        out_shape=jax.ShapeDtypeStruct((N, DP), jnp.float32),
    )(z, degp, degp)


def _scale2_body(p0_ref, p1_ref, d0_ref, d1_ref, o_ref):
    deg = 1.0 + d0_ref[0, :, 0:1] + d1_ref[0, :, 0:1]
    o_ref[...] = (p0_ref[0] + p1_ref[0]) / deg


def _scale2(sp, degp):
    return pl.pallas_call(
        _scale2_body,
        grid=(NB,),
        in_specs=[
            pl.BlockSpec((1, BR, DP), lambda i: (0, i, 0)),
            pl.BlockSpec((1, BR, DP), lambda i: (1, i, 0)),
            pl.BlockSpec((1, BR, 16), lambda i: (0, i, 0)),
            pl.BlockSpec((1, BR, 16), lambda i: (1, i, 0)),
        ],
        out_specs=pl.BlockSpec((BR, DP), lambda i: (i, 0)),
        out_shape=jax.ShapeDtypeStruct((N, DP), jnp.float32),
    )(sp, sp, degp, degp)


def _final_body(p0_ref, p1_ref, d0_ref, d1_ref, b_ref, o_ref):
    deg = 1.0 + d0_ref[0, :, 0:1] + d1_ref[0, :, 0:1]
    logits = (p0_ref[0] + p1_ref[0]) * lax.rsqrt(deg) + b_ref[...]
    col = lax.broadcasted_iota(jnp.int32, (BR, DP), 1)
    valid = col < C
    masked = jnp.where(valid, logits, -1e30)
    m = jnp.max(masked, axis=1, keepdims=True)
    e = jnp.where(valid, jnp.exp(logits - m), 0.0)
    lse = jnp.log(jnp.sum(e, axis=1, keepdims=True))
    o_ref[...] = logits - m - lse


def _final(sp, degp, bp):
    return pl.pallas_call(
        _final_body,
        grid=(NB,),
        in_specs=[
            pl.BlockSpec((1, BR, DP), lambda i: (0, i, 0)),
            pl.BlockSpec((1, BR, DP), lambda i: (1, i, 0)),
            pl.BlockSpec((1, BR, 16), lambda i: (0, i, 0)),
            pl.BlockSpec((1, BR, 16), lambda i: (1, i, 0)),
            pl.BlockSpec((1, DP), lambda i: (0, 0)),
        ],
        out_specs=pl.BlockSpec((BR, DP), lambda i: (i, 0)),
        out_shape=jax.ShapeDtypeStruct((N, DP), jnp.float32),
    )(sp, sp, degp, degp, bp)


def kernel(x, edge_index, W, b):
    src = edge_index[0].astype(jnp.int32)
    dst = edge_index[1].astype(jnp.int32)
    pad = jnp.zeros((NW, EPWP - EPW), jnp.int32)
    src3 = jnp.concatenate([src.reshape(NW, EPW), pad], axis=1)
    src3 = src3.reshape(NW, NCH, CH)
    dst3 = jnp.concatenate([dst.reshape(NW, EPW), pad + N], axis=1)
    dst3 = dst3.reshape(NW, NCH, CH)

    wp = jnp.zeros((DP, D), jnp.float32).at[:C].set(W)
    bp = jnp.zeros((1, DP), jnp.float32).at[0, :C].set(b)
    ones16 = jnp.ones((CH, 16), jnp.float32)
    zeros16 = jnp.zeros((NPAD, 16), jnp.float32)
    zerosdp = jnp.zeros((N, DP), jnp.float32)

    degp = _deg_kernel(dst3, ones16, zeros16)
    z = _matmul(x, wp)
    g1 = _scale1(z, degp)
    s1p = _hop_kernel(g1, src3, dst3, zerosdp)
    g2 = _scale2(s1p, degp)
    s2p = _hop_kernel(g2, src3, dst3, zerosdp)
    out = _final(s2p, degp, bp)
    return out[:, :C]


# speedup vs baseline: 1.7738x; 1.7738x over previous
import functools

import jax
import jax.numpy as jnp
from jax import lax
from jax.experimental import pallas as pl
from jax.experimental.pallas import tpu as pltpu
from jax.experimental.pallas import tpu_sc as plsc

N = 10000
E = 320000
D = 128
C = 40
DP = 48
NC = 2
NS = 16
NW = NC * NS
EPW = E // NW
CH = 128
NCH = 80
EPWP = CH * NCH
NPAD = N + 8
RING = 8
GDEPTH = 6
NPS = N // NS
BR = 1000
NB = N // BR

_MESH = plsc.VectorSubcoreMesh(core_axis_name="c", subcore_axis_name="s")
_SC_PARAMS = pltpu.CompilerParams(use_tc_tiling_on_sc=False)


def _deg_body(dst_hbm, ones_hbm, zeros_hbm, out_hbm, idx_d, ones_v, acc,
              sem, dsem):
    c = lax.axis_index("c")
    s = lax.axis_index("s")

    pltpu.sync_copy(zeros_hbm.at[pl.ds(s * NPS, NPS)],
                    acc.at[pl.ds(s * NPS, NPS)])
    pltpu.sync_copy(ones_hbm, ones_v)
    wid = s * NC + c
    pltpu.sync_copy(dst_hbm.at[wid], idx_d)
    plsc.subcore_barrier()

    @pl.loop(0, NCH, step=8)
    def _(j):
        for b in range(8):
            pltpu.async_copy(ones_v, acc.at[idx_d.at[j + b]], sem, add=True)
        for b in range(8):
            pltpu.make_async_copy(ones_v, acc.at[idx_d.at[j + b]], sem).wait()

    plsc.subcore_barrier()
    pltpu.sync_copy(acc.at[pl.ds(s * NPS, NPS)],
                    out_hbm.at[c, pl.ds(s * NPS, NPS)])


@functools.partial(
    pl.kernel,
    out_type=jax.ShapeDtypeStruct((NC, N, 16), jnp.float32),
    mesh=_MESH,
    scratch_types=[
        pltpu.VMEM((NCH, CH), jnp.int32),
        pltpu.VMEM((CH, 16), jnp.float32),
        pltpu.VMEM_SHARED((NPAD, 16), jnp.float32),
        pltpu.SemaphoreType.DMA,
        pltpu.SemaphoreType.DMA,
    ],
    compiler_params=_SC_PARAMS,
)
def _deg_kernel(dst_hbm, ones_hbm, zeros_hbm, out_hbm, idx_d, ones_v, acc,
                sem, dsem):
    _deg_body(dst_hbm, ones_hbm, zeros_hbm, out_hbm, idx_d, ones_v, acc,
              sem, dsem)


def _hop_body(g_hbm, src_hbm, dst_hbm, zeros_hbm, out_hbm,
              idx_s, idx_d, rows, acc, gs, gsems, ssems, dsem):
    c = lax.axis_index("c")
    s = lax.axis_index("s")

    pltpu.sync_copy(g_hbm.at[pl.ds(s * NPS, NPS)], gs.at[pl.ds(s * NPS, NPS)])

    @pl.when(c == 0)
    def _():
        pltpu.sync_copy(g_hbm.at[pl.ds(s * NPS, NPS)],
                        acc.at[pl.ds(s * NPS, NPS)])

    @pl.when(c != 0)
    def _():
        pltpu.sync_copy(zeros_hbm.at[pl.ds(s * NPS, NPS)],
                        acc.at[pl.ds(s * NPS, NPS)])

    wid = s * NC + c
    pltpu.sync_copy(src_hbm.at[wid], idx_s)
    pltpu.sync_copy(dst_hbm.at[wid], idx_d)
    plsc.subcore_barrier()

    for m in range(GDEPTH):
        pltpu.async_copy(gs.at[idx_s.at[m]], rows[m], gsems[m])

    @pl.loop(0, NCH, step=RING)
    def _(j):
        for r in range(RING):
            k = j + r
            b2 = (r + GDEPTH) % RING

            @pl.when((k >= RING - GDEPTH) & (k + GDEPTH < NCH))
            def _():
                pltpu.make_async_copy(rows[b2], acc.at[idx_d.at[k]],
                                      ssems[b2]).wait()

            @pl.when(k + GDEPTH < NCH)
            def _():
                pltpu.async_copy(gs.at[idx_s.at[k + GDEPTH]], rows[b2],
                                 gsems[b2])

            pltpu.make_async_copy(gs.at[idx_s.at[k]], rows[r],
                                  gsems[r]).wait()
            pltpu.async_copy(rows[r], acc.at[idx_d.at[k]], ssems[r],
                             add=True)

    for r in range(RING):
        pltpu.make_async_copy(rows[r], acc.at[idx_d.at[r]], ssems[r]).wait()

    plsc.subcore_barrier()
    pltpu.sync_copy(acc.at[pl.ds(s * NPS, NPS)],
                    out_hbm.at[c, pl.ds(s * NPS, NPS)])


@functools.partial(
    pl.kernel,
    out_type=jax.ShapeDtypeStruct((NC, N, DP), jnp.float32),
    mesh=_MESH,
    scratch_types=[
        pltpu.VMEM((NCH, CH), jnp.int32),
        pltpu.VMEM((NCH, CH), jnp.int32),
        [pltpu.VMEM((CH, DP), jnp.float32)] * RING,
        pltpu.VMEM_SHARED((NPAD, DP), jnp.float32),
        pltpu.VMEM_SHARED((N, DP), jnp.float32),
        [pltpu.SemaphoreType.DMA] * RING,
        [pltpu.SemaphoreType.DMA] * RING,
        pltpu.SemaphoreType.DMA,
    ],
    compiler_params=_SC_PARAMS,
)
def _hop_kernel(g_hbm, src_hbm, dst_hbm, zeros_hbm, out_hbm,
                idx_s, idx_d, rows, acc, gs, gsems, ssems, dsem):
    _hop_body(g_hbm, src_hbm, dst_hbm, zeros_hbm, out_hbm,
              idx_s, idx_d, rows, acc, gs, gsems, ssems, dsem)


def _mm_body(x_ref, w_ref, o_ref):
    o_ref[...] = lax.dot_general(
        x_ref[...], w_ref[...], (((1,), (1,)), ((), ())),
        preferred_element_type=jnp.float32)


def _matmul(x, wp):
    return pl.pallas_call(
        _mm_body,
        grid=(NB,),
        in_specs=[
            pl.BlockSpec((BR, D), lambda i: (i, 0)),
            pl.BlockSpec((DP, D), lambda i: (0, 0)),
        ],
        out_specs=pl.BlockSpec((BR, DP), lambda i: (i, 0)),
        out_shape=jax.ShapeDtypeStruct((N, DP), jnp.float32),
    )(x, wp)


def _scale1_body(z_ref, d0_ref, d1_ref, o_ref):
    deg = 1.0 + d0_ref[0, :, 0:1] + d1_ref[0, :, 0:1]
    o_ref[...] = z_ref[...] * lax.rsqrt(deg)


def _scale1(z, degp):
    return pl.pallas_call(
        _scale1_body,
        grid=(NB,),
        in_specs=[
            pl.BlockSpec((BR, DP), lambda i: (i, 0)),
            pl.BlockSpec((1, BR, 16), lambda i: (0, i, 0)),
            pl.BlockSpec((1, BR, 16), lambda i: (1, i, 0)),
        ],
        out_specs=pl.BlockSpec((BR, DP), lambda i: (i, 0)),
        out_shape=jax.ShapeDtypeStruct((N, DP), jnp.float32),
    )(z, degp, degp)


def _scale2_body(p0_ref, p1_ref, d0_ref, d1_ref, o_ref):
    deg = 1.0 + d0_ref[0, :, 0:1] + d1_ref[0, :, 0:1]
    o_ref[...] = (p0_ref[0] + p1_ref[0]) / deg


def _scale2(sp, degp):
    return pl.pallas_call(
        _scale2_body,
        grid=(NB,),
        in_specs=[
            pl.BlockSpec((1, BR, DP), lambda i: (0, i, 0)),
            pl.BlockSpec((1, BR, DP), lambda i: (1, i, 0)),
            pl.BlockSpec((1, BR, 16), lambda i: (0, i, 0)),
            pl.BlockSpec((1, BR, 16), lambda i: (1, i, 0)),
        ],
        out_specs=pl.BlockSpec((BR, DP), lambda i: (i, 0)),
        out_shape=jax.ShapeDtypeStruct((N, DP), jnp.float32),
    )(sp, sp, degp, degp)


def _final_body(p0_ref, p1_ref, d0_ref, d1_ref, b_ref, o_ref):
    deg = 1.0 + d0_ref[0, :, 0:1] + d1_ref[0, :, 0:1]
    logits = (p0_ref[0] + p1_ref[0]) * lax.rsqrt(deg) + b_ref[...]
    col = lax.broadcasted_iota(jnp.int32, (BR, DP), 1)
    valid = col < C
    masked = jnp.where(valid, logits, -1e30)
    m = jnp.max(masked, axis=1, keepdims=True)
    e = jnp.where(valid, jnp.exp(logits - m), 0.0)
    lse = jnp.log(jnp.sum(e, axis=1, keepdims=True))
    o_ref[...] = logits - m - lse


def _final(sp, degp, bp):
    return pl.pallas_call(
        _final_body,
        grid=(NB,),
        in_specs=[
            pl.BlockSpec((1, BR, DP), lambda i: (0, i, 0)),
            pl.BlockSpec((1, BR, DP), lambda i: (1, i, 0)),
            pl.BlockSpec((1, BR, 16), lambda i: (0, i, 0)),
            pl.BlockSpec((1, BR, 16), lambda i: (1, i, 0)),
            pl.BlockSpec((1, DP), lambda i: (0, 0)),
        ],
        out_specs=pl.BlockSpec((BR, DP), lambda i: (i, 0)),
        out_shape=jax.ShapeDtypeStruct((N, DP), jnp.float32),
    )(sp, sp, degp, degp, bp)


def kernel(x, edge_index, W, b):
    src = edge_index[0].astype(jnp.int32)
    dst = edge_index[1].astype(jnp.int32)
    pad = jnp.zeros((NW, EPWP - EPW), jnp.int32)
    src3 = jnp.concatenate([src.reshape(NW, EPW), pad], axis=1)
    src3 = src3.reshape(NW, NCH, CH)
    dst3 = jnp.concatenate([dst.reshape(NW, EPW), pad + N], axis=1)
    dst3 = dst3.reshape(NW, NCH, CH)

    wp = jnp.zeros((DP, D), jnp.float32).at[:C].set(W)
    bp = jnp.zeros((1, DP), jnp.float32).at[0, :C].set(b)
    ones16 = jnp.ones((CH, 16), jnp.float32)
    zeros16 = jnp.zeros((NPAD, 16), jnp.float32)
    zerosdp = jnp.zeros((N, DP), jnp.float32)

    degp = _deg_kernel(dst3, ones16, zeros16)
    z = _matmul(x, wp)
    g1 = _scale1(z, degp)
    s1p = _hop_kernel(g1, src3, dst3, zerosdp)
    g2 = _scale2(s1p, degp)
    s2p = _hop_kernel(g2, src3, dst3, zerosdp)
    out = _final(s2p, degp, bp)
    return out[:, :C]
